# Initial kernel scaffold; baseline (speedup 1.0000x reference)
import jax
import jax.numpy as jnp
from jax import lax
from jax.experimental import pallas as pl
from jax.experimental.pallas import tpu as pltpu
from jax.experimental.pallas import tpu_sc as plsc

NUM_FUNCS = 32
TABLE_ROWS = 256
LANES = 16

_B = 16384 * 100
_NW = 32
_PER_W = _B // _NW
_CHUNK = 1024
_NCHUNK = _PER_W // _CHUNK


def _rbf_kernel(table_hbm, idx_hbm, out_hbm, table_v, idx_v, out_v):
    wid = lax.axis_index("s") * 2 + lax.axis_index("c")
    wstart = wid * _PER_W

    pltpu.sync_copy(table_hbm, table_v)

    lane32 = lax.iota(jnp.int32, LANES) * NUM_FUNCS

    def chunk_body(g, carry):
        base = wstart + g * _CHUNK
        pltpu.sync_copy(idx_hbm.at[pl.ds(base, _CHUNK)], idx_v)

        def group_body(i, c2):
            r = idx_v[pl.ds(i * LANES, LANES)]
            r = jnp.minimum(jnp.maximum(r, 0), TABLE_ROWS - 1)
            r32 = r * NUM_FUNCS
            sbase = lane32 + i * (LANES * NUM_FUNCS)
            for j in range(NUM_FUNCS):
                vals = plsc.load_gather(table_v, [r32 + j])
                plsc.store_scatter(out_v, [sbase + j], vals)
            return c2

        lax.fori_loop(0, _CHUNK // LANES, group_body, 0)
        pltpu.sync_copy(out_v, out_hbm.at[pl.ds(base * NUM_FUNCS,
                                                _CHUNK * NUM_FUNCS)])
        return carry

    lax.fori_loop(0, _NCHUNK, chunk_body, 0)


def kernel(tensor, int_to_feat_matrix, extra_embeddings):
    orig_shape = tensor.shape
    table = jnp.concatenate([int_to_feat_matrix, extra_embeddings], axis=0)
    table_flat = table.reshape(-1)
    idx_flat = tensor.reshape(-1).astype(jnp.int32)

    mesh = plsc.VectorSubcoreMesh(core_axis_name="c", subcore_axis_name="s")
    run = pl.kernel(
        _rbf_kernel,
        mesh=mesh,
        out_type=jax.ShapeDtypeStruct((_B * NUM_FUNCS,), jnp.float32),
        scratch_types=[
            pltpu.VMEM((TABLE_ROWS * NUM_FUNCS,), jnp.float32),
            pltpu.VMEM((_CHUNK,), jnp.int32),
            pltpu.VMEM((_CHUNK * NUM_FUNCS,), jnp.float32),
        ],
    )
    out_flat = run(table_flat, idx_flat)
    return out_flat.reshape(*orig_shape[:-1], orig_shape[-1] * NUM_FUNCS)


# speedup vs baseline: 5.0705x; 5.0705x over previous
import jax
import jax.numpy as jnp
from jax import lax
from jax.experimental import pallas as pl
from jax.experimental.pallas import tpu as pltpu
from jax.experimental.pallas import tpu_sc as plsc

NUM_FUNCS = 32
TABLE_ROWS = 256
LANES = 16

_B = 16384 * 100
_NW = 32
_PER_W = _B // _NW
_CHUNK = 1024
_NCHUNK = _PER_W // _CHUNK


def _rbf_kernel(table_hbm, idx_hbm, out_hbm, table_v, idx_v, out_v):
    wid = lax.axis_index("s") * 2 + lax.axis_index("c")
    wstart = wid * _PER_W

    pltpu.sync_copy(table_hbm, table_v)

    lane32 = lax.iota(jnp.int32, LANES) * NUM_FUNCS

    def chunk_body(g, carry):
        base = wstart + g * _CHUNK
        pltpu.sync_copy(idx_hbm.at[pl.ds(base, _CHUNK)], idx_v)

        def group_body(i, c2):
            r = idx_v[pl.ds(i * LANES, LANES)]
            r = jnp.minimum(jnp.maximum(r, 0), TABLE_ROWS - 1)
            r32 = r * NUM_FUNCS
            sbase = lane32 + i * (LANES * NUM_FUNCS)
            for j in range(NUM_FUNCS):
                vals = plsc.load_gather(table_v, [r32 + j])
                plsc.store_scatter(out_v, [sbase + j], vals)
            return c2

        lax.fori_loop(0, _CHUNK // LANES, group_body, 0)
        pltpu.sync_copy(out_v, out_hbm.at[pl.ds(base * NUM_FUNCS,
                                                _CHUNK * NUM_FUNCS)])
        return carry

    lax.fori_loop(0, _NCHUNK, chunk_body, 0)


def kernel(tensor, int_to_feat_matrix, extra_embeddings):
    orig_shape = tensor.shape
    table = jnp.concatenate([int_to_feat_matrix, extra_embeddings], axis=0)
    table_flat = table.reshape(-1)
    idx_flat = tensor.reshape(-1).astype(jnp.int32)

    mesh = plsc.VectorSubcoreMesh(core_axis_name="c", subcore_axis_name="s")
    run = pl.kernel(
        _rbf_kernel,
        mesh=mesh,
        out_type=jax.ShapeDtypeStruct((_B * NUM_FUNCS,), jnp.float32),
        scratch_types=[
            pltpu.VMEM((TABLE_ROWS * NUM_FUNCS,), jnp.float32),
            pltpu.VMEM((_CHUNK,), jnp.int32),
            pltpu.VMEM((_CHUNK * NUM_FUNCS,), jnp.float32),
        ],
        compiler_params=pltpu.CompilerParams(needs_layout_passes=False),
    )
    out_flat = run(table_flat, idx_flat)
    return out_flat.reshape(*orig_shape[:-1], orig_shape[-1] * NUM_FUNCS)


# speedup vs baseline: 30.4968x; 6.0145x over previous
import jax
import jax.numpy as jnp
from jax import lax
from jax.experimental import pallas as pl
from jax.experimental.pallas import tpu as pltpu
from jax.experimental.pallas import tpu_sc as plsc

NUM_FUNCS = 32
TABLE_ROWS = 256
LANES = 16

_B = 16384 * 100
_NW = 32
_PER_W = _B // _NW
_CHUNK = 1024
_NCHUNK = _PER_W // _CHUNK
_NBUF = 2


def _rbf_kernel(table_hbm, idx_hbm, out_hbm, table_v, idx_v0, idx_v1,
                out_v0, out_v1, si0, si1, so0, so1):
    wid = lax.axis_index("s") * 2 + lax.axis_index("c")
    wstart = wid * _PER_W
    idx_v = [idx_v0, idx_v1]
    out_v = [out_v0, out_v1]
    si = [si0, si1]
    so = [so0, so1]

    pltpu.sync_copy(table_hbm, table_v)

    lane = lax.iota(jnp.int32, LANES)
    lane32 = lane * NUM_FUNCS

    def in_copy(b, g):
        base = wstart + g * _CHUNK
        return pltpu.make_async_copy(
            idx_hbm.at[pl.ds(base, _CHUNK)], idx_v[b], si[b])

    def out_copy(b, g):
        base = wstart + g * _CHUNK
        return pltpu.make_async_copy(
            out_v[b], out_hbm.at[pl.ds(base * NUM_FUNCS,
                                          _CHUNK * NUM_FUNCS)], so[b])

    for b in range(_NBUF):
        in_copy(b, b).start()

    def outer_body(o, carry):
        for b in range(_NBUF):
            g = o * _NBUF + b
            in_copy(b, g).wait()

            @pl.when(o > 0)
            def _wait_out():
                out_copy(b, g - _NBUF).wait()

            def group_body(i, c2):
                r = idx_v[b][pl.ds(i * LANES, LANES)]
                r = jnp.minimum(jnp.maximum(r, 0), TABLE_ROWS - 1)
                r32 = r * NUM_FUNCS
                sblane = lane32 + i * (LANES * NUM_FUNCS)

                @plsc.parallel_loop(0, NUM_FUNCS, unroll=8)
                def kbody(k):
                    jv = (lane + k) & (NUM_FUNCS - 1)
                    vals = plsc.load_gather(table_v, [r32 + jv])
                    plsc.store_scatter(out_v[b], [sblane + jv], vals)

                return c2

            lax.fori_loop(0, _CHUNK // LANES, group_body, 0)
            out_copy(b, g).start()

            @pl.when(o < _NCHUNK // _NBUF - 1)
            def _prefetch():
                in_copy(b, g + _NBUF).start()

        return carry

    lax.fori_loop(0, _NCHUNK // _NBUF, outer_body, 0)
    for b in range(_NBUF):
        out_copy(b, _NCHUNK - _NBUF + b).wait()


def kernel(tensor, int_to_feat_matrix, extra_embeddings):
    orig_shape = tensor.shape
    table = jnp.concatenate([int_to_feat_matrix, extra_embeddings], axis=0)
    table_flat = table.reshape(-1)
    idx_flat = tensor.reshape(-1).astype(jnp.int32)

    mesh = plsc.VectorSubcoreMesh(core_axis_name="c", subcore_axis_name="s")
    run = pl.kernel(
        _rbf_kernel,
        mesh=mesh,
        out_type=jax.ShapeDtypeStruct((_B * NUM_FUNCS,), jnp.float32),
        scratch_types=[
            pltpu.VMEM((TABLE_ROWS * NUM_FUNCS,), jnp.float32),
            pltpu.VMEM((_CHUNK,), jnp.int32),
            pltpu.VMEM((_CHUNK,), jnp.int32),
            pltpu.VMEM((_CHUNK * NUM_FUNCS,), jnp.float32),
            pltpu.VMEM((_CHUNK * NUM_FUNCS,), jnp.float32),
            pltpu.SemaphoreType.DMA,
            pltpu.SemaphoreType.DMA,
            pltpu.SemaphoreType.DMA,
            pltpu.SemaphoreType.DMA,
        ],
        compiler_params=pltpu.CompilerParams(needs_layout_passes=False),
    )
    out_flat = run(table_flat, idx_flat)
    return out_flat.reshape(*orig_shape[:-1], orig_shape[-1] * NUM_FUNCS)


# speedup vs baseline: 62.2263x; 2.0404x over previous
import jax
import jax.numpy as jnp
from jax import lax
from jax.experimental import pallas as pl
from jax.experimental.pallas import tpu as pltpu
from jax.experimental.pallas import tpu_sc as plsc

NUM_FUNCS = 32
TABLE_ROWS = 256
LANES = 16

_ROWS = 16384
_COLS = 100
_OUT_COLS = _COLS * NUM_FUNCS
_NW = 32
_ROWS_W = _ROWS // _NW
_CROWS = 16
_CHUNK = _CROWS * _COLS
_NCHUNK = _ROWS_W // _CROWS
_NBUF = 2


def _rbf_kernel(table_hbm, idx_hbm, out_hbm, table_v, idx_v0, idx_v1,
                out_v0, out_v1, si0, si1, so0, so1):
    wid = lax.axis_index("s") * 2 + lax.axis_index("c")
    wrow = wid * _ROWS_W
    idx_v = [idx_v0, idx_v1]
    out_v = [out_v0, out_v1]
    si = [si0, si1]
    so = [so0, so1]

    pltpu.sync_copy(table_hbm, table_v)

    lane = lax.iota(jnp.int32, LANES)

    def in_copy(b, g):
        r0 = wrow + g * _CROWS
        return pltpu.make_async_copy(
            idx_hbm.at[pl.ds(r0, _CROWS), :], idx_v[b], si[b])

    def out_copy(b, g):
        r0 = wrow + g * _CROWS
        return pltpu.make_async_copy(
            out_v[b], out_hbm.at[pl.ds(r0, _CROWS), :], so[b])

    for b in range(_NBUF):
        in_copy(b, b).start()

    def outer_body(o, carry):
        for b in range(_NBUF):
            g = o * _NBUF + b
            in_copy(b, g).wait()

            @pl.when(o > 0)
            def _wait_out():
                out_copy(b, g - _NBUF).wait()

            def group_body(i, c2):
                e = i * LANES + lane
                ri = (e * 5243) >> 19
                c = e - ri * _COLS
                r = plsc.load_gather(idx_v[b], [ri, c])
                r = jnp.minimum(jnp.maximum(r, 0), TABLE_ROWS - 1)
                r32 = r * NUM_FUNCS
                c32 = c * NUM_FUNCS

                @plsc.parallel_loop(0, NUM_FUNCS, unroll=8)
                def kbody(k):
                    jv = (lane + k) & (NUM_FUNCS - 1)
                    vals = plsc.load_gather(table_v, [r32 + jv])
                    plsc.store_scatter(out_v[b], [ri, c32 + jv], vals)

                return c2

            lax.fori_loop(0, _CHUNK // LANES, group_body, 0)
            out_copy(b, g).start()

            @pl.when(o < _NCHUNK // _NBUF - 1)
            def _prefetch():
                in_copy(b, g + _NBUF).start()

        return carry

    lax.fori_loop(0, _NCHUNK // _NBUF, outer_body, 0)
    for b in range(_NBUF):
        out_copy(b, _NCHUNK - _NBUF + b).wait()


def kernel(tensor, int_to_feat_matrix, extra_embeddings):
    orig_shape = tensor.shape
    table = jnp.concatenate([int_to_feat_matrix, extra_embeddings], axis=0)
    table_flat = table.reshape(-1)
    idx = tensor.astype(jnp.int32)

    mesh = plsc.VectorSubcoreMesh(core_axis_name="c", subcore_axis_name="s")
    run = pl.kernel(
        _rbf_kernel,
        mesh=mesh,
        out_type=jax.ShapeDtypeStruct((_ROWS, _OUT_COLS), jnp.float32),
        scratch_types=[
            pltpu.VMEM((TABLE_ROWS * NUM_FUNCS,), jnp.float32),
            pltpu.VMEM((_CROWS, _COLS), jnp.int32),
            pltpu.VMEM((_CROWS, _COLS), jnp.int32),
            pltpu.VMEM((_CROWS, _OUT_COLS), jnp.float32),
            pltpu.VMEM((_CROWS, _OUT_COLS), jnp.float32),
            pltpu.SemaphoreType.DMA,
            pltpu.SemaphoreType.DMA,
            pltpu.SemaphoreType.DMA,
            pltpu.SemaphoreType.DMA,
        ],
        compiler_params=pltpu.CompilerParams(needs_layout_passes=False),
    )
    out = run(table_flat, idx)
    return out.reshape(*orig_shape[:-1], orig_shape[-1] * NUM_FUNCS)
